# Initial kernel scaffold; baseline (speedup 1.0000x reference)
import jax
import jax.numpy as jnp
from jax.experimental import pallas as pl


def kernel(w1, b1, w2, b2, wl, bl, whd, bhd, x):
    raise NotImplementedError("write your pallas kernel here")


# speedup vs baseline: 3.0010x; 3.0010x over previous
import jax
import jax.numpy as jnp
from jax.experimental import pallas as pl
from jax.experimental.pallas import tpu as pltpu

_TB = 16


def _fused_kernel(x_ref, w1b_ref, b1_ref, w2_ref, b2_ref,
                  wl_ref, bl_ref, whd_ref, bhd_ref, out_ref):
    f32 = jnp.float32
    bf16 = jnp.bfloat16
    tb = x_ref.shape[0]

    xb2 = x_ref[...]
    lhs1 = jnp.concatenate([xb2[:, 0:20, :], xb2[:, 1:21, :]],
                           axis=2)
    lhs1 = lhs1.reshape(tb * 20, 112)
    o1 = jnp.dot(lhs1, w1b_ref[...], preferred_element_type=f32)
    o1 = o1.reshape(tb, 20, 1664)
    ph = jnp.maximum(o1[..., :832], o1[..., 832:])
    v1 = ph.reshape(tb, 20, 13, 64)
    a1 = jnp.maximum(v1[..., :32], v1[..., 32:])
    a1 = jnp.maximum(a1 + b1_ref[...], 0.0)

    parts = [a1[:, i:i + 18, j:j + 11, :]
             for i in range(3) for j in range(3)]
    lhs2 = jnp.concatenate(parts, axis=3)
    lhs2 = lhs2.reshape(tb * 198, 288).astype(bf16)
    o2 = jnp.dot(lhs2, w2_ref[...], preferred_element_type=f32)
    o2 = o2.reshape(tb, 9, 2, 11, 64)
    p2 = jnp.maximum(o2[:, :, 0], o2[:, :, 1])
    p2 = jnp.maximum(p2 + b2_ref[...], 0.0)
    fw = jnp.concatenate(
        [jnp.maximum(p2[:, :, 2 * k, :], p2[:, :, 2 * k + 1, :])
         for k in range(5)], axis=-1)

    flat = jnp.concatenate([fw[:, q, :] for q in range(9)],
                           axis=-1).astype(bf16)
    h = jnp.dot(flat, wl_ref[...], preferred_element_type=f32) + bl_ref[...]
    z = jnp.dot(h, whd_ref[...], preferred_element_type=f32) + bhd_ref[...]
    out_ref[...] = z


def _pack_conv1_band(w1):
    wt = w1.reshape(3, 3, 32)
    r = jnp.arange(4)
    w = jnp.arange(28)
    dh = jnp.arange(2)
    wo = jnp.arange(26)
    kh = r[:, None] - dh[None, :]
    kw = w[:, None] - wo[None, :]
    mask = ((kh >= 0) & (kh < 3))[:, None, :, None] & \
           ((kw >= 0) & (kw < 3))[None, :, None, :]
    khc = jnp.clip(kh, 0, 2)
    kwc = jnp.clip(kw, 0, 2)
    vals = wt[kwc[None, :, None, :], khc[:, None, :, None], :]
    vals = jnp.where(mask[..., None], vals, 0.0)
    return vals.reshape(112, 1664).astype(jnp.float32)


@jax.jit
def _forward(w1, b1, w2, b2, wl, bl, whd, bhd, x):
    N = x.shape[0]
    tb = _TB if N >= _TB else N
    n_pad = ((N + tb - 1) // tb) * tb
    xs = x.reshape(N, 21, 56)
    if n_pad != N:
        xs = jnp.concatenate(
            [xs, jnp.zeros((n_pad - N, 21, 56), xs.dtype)], axis=0)
    w1b = _pack_conv1_band(w1)

    out = pl.pallas_call(
        _fused_kernel,
        out_shape=jax.ShapeDtypeStruct((n_pad, 128), jnp.float32),
        grid=(n_pad // tb,),
        in_specs=[
            pl.BlockSpec((tb, 21, 56), lambda n: (n, 0, 0)),
            pl.BlockSpec((112, 1664), lambda n: (0, 0)),
            pl.BlockSpec((1, 32), lambda n: (0, 0)),
            pl.BlockSpec((288, 64), lambda n: (0, 0)),
            pl.BlockSpec((1, 64), lambda n: (0, 0)),
            pl.BlockSpec((2880, 128), lambda n: (0, 0)),
            pl.BlockSpec((1, 128), lambda n: (0, 0)),
            pl.BlockSpec((128, 128), lambda n: (0, 0)),
            pl.BlockSpec((1, 128), lambda n: (0, 0)),
        ],
        out_specs=pl.BlockSpec((tb, 128), lambda n: (n, 0)),
        compiler_params=pltpu.CompilerParams(
            dimension_semantics=("parallel",),
            vmem_limit_bytes=64 * 1024 * 1024,
        ),
    )(xs, w1b, b1, w2, b2, wl, bl, whd, bhd)

    logits = out[:N, :20]
    return logits[:, :10], logits[:, 10:]


def kernel(w1, b1, w2, b2, wl, bl, whd, bhd, x):
    return _forward(w1, b1, w2, b2, wl, bl, whd, bhd, x)
